# baseline (device time: 29574 ns/iter reference)
import jax
import jax.numpy as jnp
from jax import lax
from jax.experimental import pallas as pl
from jax.experimental.pallas import tpu as pltpu

B, SQ, SKV, D = 2, 128, 128, 512
HQ_LOC, DH = 8, 64


def kernel(x, Wq, Wo, K_ext, V_ext):
    me = lax.axis_index("i")
    K_loc = lax.dynamic_slice_in_dim(K_ext, me * HQ_LOC, HQ_LOC, axis=2)
    V_loc = lax.dynamic_slice_in_dim(V_ext, me * HQ_LOC, HQ_LOC, axis=2)
    K2 = jnp.transpose(K_loc, (0, 2, 1, 3)).reshape(B * HQ_LOC, SKV, DH)
    V2 = jnp.transpose(V_loc, (0, 2, 1, 3)).reshape(B * HQ_LOC, SKV, DH)

    def body(x_ref, wq_ref, wo_ref, k_ref, v_ref, out_ref,
             attn_ref, acc_ref, recv_ref, send_sems, recv_sems):
        my_pos = lax.axis_index("i")

        x2 = x_ref[:].reshape(B * SQ, D)
        q2 = jnp.dot(x2, wq_ref[:], preferred_element_type=jnp.float32)
        for b in range(B):
            for h in range(HQ_LOC):
                idx = b * HQ_LOC + h
                q = q2[b * SQ:(b + 1) * SQ, h * DH:(h + 1) * DH]
                k = k_ref[idx]
                v = v_ref[idx]
                s = lax.dot_general(
                    q, k, (((1,), (1,)), ((), ())),
                    preferred_element_type=jnp.float32,
                ) * 0.125
                m = jnp.max(s, axis=1, keepdims=True)
                p = jnp.exp(s - m)
                l = jnp.sum(p, axis=1, keepdims=True)
                o = jnp.dot(p, v, preferred_element_type=jnp.float32) / l
                attn_ref[b * SQ:(b + 1) * SQ, h * DH:(h + 1) * DH] = o
        acc_ref[:] = jnp.dot(attn_ref[:], wo_ref[:],
                             preferred_element_type=jnp.float32)

        partners = [my_pos ^ 1, 3 - my_pos]
        for step in range(2):
            rdma = pltpu.make_async_remote_copy(
                src_ref=acc_ref,
                dst_ref=recv_ref.at[step],
                send_sem=send_sems.at[step],
                recv_sem=recv_sems.at[step],
                device_id=(partners[step],),
                device_id_type=pl.DeviceIdType.MESH,
            )
            rdma.start()
            rdma.wait()
            acc_ref[:] = acc_ref[:] + recv_ref[step]

        out_ref[:] = acc_ref[:].reshape(B, SQ, D)

    return pl.pallas_call(
        body,
        out_shape=jax.ShapeDtypeStruct((B, SQ, D), jnp.float32),
        in_specs=[pl.BlockSpec(memory_space=pltpu.VMEM)] * 5,
        out_specs=pl.BlockSpec(memory_space=pltpu.VMEM),
        scratch_shapes=[
            pltpu.VMEM((B * SQ, D), jnp.float32),
            pltpu.VMEM((B * SQ, D), jnp.float32),
            pltpu.VMEM((2, B * SQ, D), jnp.float32),
            pltpu.SemaphoreType.DMA((2,)),
            pltpu.SemaphoreType.DMA((2,)),
        ],
    )(x, Wq, Wo, K2, V2)


# device time: 24003 ns/iter; 1.2321x vs baseline; 1.2321x over previous
import jax
import jax.numpy as jnp
from jax import lax
from jax.experimental import pallas as pl
from jax.experimental.pallas import tpu as pltpu

B, SQ, SKV, D = 2, 128, 128, 512
HQ_LOC, DH = 8, 64


def kernel(x, Wq, Wo, K_ext, V_ext):
    me = lax.axis_index("i")
    K_loc = lax.dynamic_slice_in_dim(K_ext, me * HQ_LOC, HQ_LOC, axis=2)
    V_loc = lax.dynamic_slice_in_dim(V_ext, me * HQ_LOC, HQ_LOC, axis=2)
    K2 = jnp.transpose(K_loc, (0, 2, 1, 3)).reshape(B * HQ_LOC, SKV, DH)
    V2 = jnp.transpose(V_loc, (0, 2, 1, 3)).reshape(B * HQ_LOC, SKV, DH)

    def body(x_ref, wq_ref, wo_ref, k_ref, v_ref, out_ref,
             attn_ref, acc_ref, recv_ref, send_sems, recv_sems):
        my_pos = lax.axis_index("i")
        partners = [my_pos ^ 1, 3 - my_pos]


        def rows(c):
            return pl.ds(c * SQ, SQ)

        def compute_chunk(c):
            q2 = jnp.dot(x_ref[c], wq_ref[:],
                         preferred_element_type=jnp.float32)
            for h in range(HQ_LOC):
                idx = c * HQ_LOC + h
                q = q2[:, h * DH:(h + 1) * DH]
                s = lax.dot_general(
                    q, k_ref[idx], (((1,), (1,)), ((), ())),
                    preferred_element_type=jnp.float32,
                ) * 0.125
                m = jnp.max(s, axis=1, keepdims=True)
                p = jnp.exp(s - m)
                l = jnp.sum(p, axis=1, keepdims=True)
                o = jnp.dot(p, v_ref[idx],
                            preferred_element_type=jnp.float32) / l
                attn_ref[rows(c), h * DH:(h + 1) * DH] = o
            acc_ref[rows(c), :] = jnp.dot(
                attn_ref[rows(c), :], wo_ref[:],
                preferred_element_type=jnp.float32)

        def make_rdma(c, step):
            return pltpu.make_async_remote_copy(
                src_ref=acc_ref.at[rows(c)],
                dst_ref=recv_ref.at[step, c],
                send_sem=send_sems.at[step, c],
                recv_sem=recv_sems.at[step, c],
                device_id=(partners[step],),
                device_id_type=pl.DeviceIdType.MESH,
            )

        compute_chunk(0)
        r00 = make_rdma(0, 0)
        r00.start()
        compute_chunk(1)
        r01 = make_rdma(1, 0)
        r01.start()

        r00.wait()
        acc_ref[rows(0), :] = acc_ref[rows(0), :] + recv_ref[0, 0]
        r10 = make_rdma(0, 1)
        r10.start()

        r01.wait()
        acc_ref[rows(1), :] = acc_ref[rows(1), :] + recv_ref[0, 1]
        r11 = make_rdma(1, 1)
        r11.start()

        r10.wait()
        out_ref[0, :, :] = acc_ref[rows(0), :] + recv_ref[1, 0]
        r11.wait()
        out_ref[1, :, :] = acc_ref[rows(1), :] + recv_ref[1, 1]

    return pl.pallas_call(
        body,
        out_shape=jax.ShapeDtypeStruct((B, SQ, D), jnp.float32),
        in_specs=[pl.BlockSpec(memory_space=pltpu.VMEM)] * 5,
        out_specs=pl.BlockSpec(memory_space=pltpu.VMEM),
        scratch_shapes=[
            pltpu.VMEM((B * SQ, D), jnp.float32),
            pltpu.VMEM((B * SQ, D), jnp.float32),
            pltpu.VMEM((2, B, SQ, D), jnp.float32),
            pltpu.SemaphoreType.DMA((2, 2)),
            pltpu.SemaphoreType.DMA((2, 2)),
        ],
    )(x, Wq, Wo, K2, V2)


# device time: 21179 ns/iter; 1.3964x vs baseline; 1.1333x over previous
import jax
import jax.numpy as jnp
from jax import lax
from jax.experimental import pallas as pl
from jax.experimental.pallas import tpu as pltpu

B, SQ, SKV, D = 2, 128, 128, 512
HQ_LOC, DH = 8, 64


def kernel(x, Wq, Wo, K_ext, V_ext):
    me = lax.axis_index("i")
    K_loc = lax.dynamic_slice_in_dim(K_ext, me * HQ_LOC, HQ_LOC, axis=2)
    V_loc = lax.dynamic_slice_in_dim(V_ext, me * HQ_LOC, HQ_LOC, axis=2)
    K2 = jnp.transpose(K_loc, (0, 2, 1, 3)).reshape(B * HQ_LOC, SKV, DH)
    V2 = jnp.transpose(V_loc, (0, 2, 1, 3)).reshape(B * HQ_LOC, SKV, DH)

    def body(x_ref, wq_ref, wo_ref, k_ref, v_ref, out_ref,
             attn_ref, acc_ref, send_ref, recv_ref, send_sems, recv_sems):
        my_pos = lax.axis_index("i")
        partners = [my_pos ^ 1, 3 - my_pos]


        def rows(c):
            return pl.ds(c * SQ, SQ)

        def compute_chunk(c):
            q2 = jnp.dot(x_ref[c], wq_ref[:],
                         preferred_element_type=jnp.float32)
            for h in range(HQ_LOC):
                idx = c * HQ_LOC + h
                q = q2[:, h * DH:(h + 1) * DH]
                s = lax.dot_general(
                    q, k_ref[idx], (((1,), (1,)), ((), ())),
                    preferred_element_type=jnp.float32,
                ) * 0.125
                m = jnp.max(s, axis=1, keepdims=True)
                p = jnp.exp(s - m)
                l = jnp.sum(p, axis=1, keepdims=True)
                o = jnp.dot(p, v_ref[idx],
                            preferred_element_type=jnp.float32) / l
                attn_ref[rows(c), h * DH:(h + 1) * DH] = o
            acc_ref[rows(c), :] = jnp.dot(
                attn_ref[rows(c), :], wo_ref[:],
                preferred_element_type=jnp.float32)

        def start_rdma(c, step):
            send_ref[step, c] = acc_ref[rows(c), :].astype(jnp.bfloat16)
            rdma = pltpu.make_async_remote_copy(
                src_ref=send_ref.at[step, c],
                dst_ref=recv_ref.at[step, c],
                send_sem=send_sems.at[step, c],
                recv_sem=recv_sems.at[step, c],
                device_id=(partners[step],),
                device_id_type=pl.DeviceIdType.MESH,
            )
            rdma.start()
            return rdma

        compute_chunk(0)
        r00 = start_rdma(0, 0)
        compute_chunk(1)
        r01 = start_rdma(1, 0)

        r00.wait()
        acc_ref[rows(0), :] = (acc_ref[rows(0), :]
                               + recv_ref[0, 0].astype(jnp.float32))
        r10 = start_rdma(0, 1)

        r01.wait()
        acc_ref[rows(1), :] = (acc_ref[rows(1), :]
                               + recv_ref[0, 1].astype(jnp.float32))
        r11 = start_rdma(1, 1)

        r10.wait()
        out_ref[0, :, :] = (acc_ref[rows(0), :]
                            + recv_ref[1, 0].astype(jnp.float32))
        r11.wait()
        out_ref[1, :, :] = (acc_ref[rows(1), :]
                            + recv_ref[1, 1].astype(jnp.float32))

    return pl.pallas_call(
        body,
        out_shape=jax.ShapeDtypeStruct((B, SQ, D), jnp.float32),
        in_specs=[pl.BlockSpec(memory_space=pltpu.VMEM)] * 5,
        out_specs=pl.BlockSpec(memory_space=pltpu.VMEM),
        scratch_shapes=[
            pltpu.VMEM((B * SQ, D), jnp.float32),
            pltpu.VMEM((B * SQ, D), jnp.float32),
            pltpu.VMEM((2, B, SQ, D), jnp.bfloat16),
            pltpu.VMEM((2, B, SQ, D), jnp.bfloat16),
            pltpu.SemaphoreType.DMA((2, 2)),
            pltpu.SemaphoreType.DMA((2, 2)),
        ],
    )(x, Wq, Wo, K2, V2)
